# baseline (speedup 1.0000x reference)
import functools

import jax
import jax.numpy as jnp
from jax import lax
from jax.experimental import pallas as pl
from jax.experimental.pallas import tpu as pltpu
from jax.experimental.pallas import tpu_sc as plsc

_TILE = 8


@functools.lru_cache(maxsize=None)
def _build(B, C, D, R):
    info = plsc.get_sparse_core_info()
    nw = info.num_cores * info.num_subcores
    while B % nw != 0:
        nw //= 2
    bpw = B // nw
    nc = info.num_cores
    nlanes = info.num_lanes

    mesh = plsc.VectorSubcoreMesh(core_axis_name="c", subcore_axis_name="s")

    @functools.partial(
        pl.kernel,
        mesh=mesh,
        out_type=jax.ShapeDtypeStruct((B, R + C, D), jnp.float32),
        scratch_types=[
            pltpu.VMEM((_TILE,), jnp.int32),
            pltpu.VMEM((C, D), jnp.float32),
            pltpu.VMEM((R + C, D), jnp.float32),
            pltpu.SemaphoreType.DMA,
            pltpu.SemaphoreType.DMA,
            pltpu.SemaphoreType.DMA,
        ],
    )
    def sc_concat(x_hbm, idx_hbm, emb_hbm, out_hbm, idx_v, x_v, o_v, gsem, isem, osem):
        wid = lax.axis_index("s") * nc + lax.axis_index("c")

        @pl.when(wid < nw)
        def _():
            base = wid * bpw
            pltpu.sync_copy(idx_hbm, idx_v)
            pltpu.async_copy(emb_hbm.at[idx_v], o_v.at[pl.ds(0, _TILE)], gsem).wait()

            def start_in(i):
                return pltpu.async_copy(x_hbm.at[base + i], x_v, isem)

            def start_out(i):
                return pltpu.async_copy(o_v, out_hbm.at[base + i], osem)

            nchunk = D // nlanes
            ngrp = (R + C) // _TILE

            def shift():
                for rr in range(R, _TILE):
                    for c in range(nchunk):
                        sl = pl.ds(c * nlanes, nlanes)
                        o_v[rr, sl] = x_v[rr - R, sl]

                @plsc.parallel_loop(1, ngrp)
                def _grp(g):
                    for rr in range(_TILE):
                        if rr < R:
                            src = 8 * (g - 1) + (_TILE - R + rr)
                        else:
                            src = 8 * g + (rr - R)
                        for c in range(nchunk):
                            sl = pl.ds(c * nlanes, nlanes)
                            o_v[8 * g + rr, sl] = x_v[src, sl]

                for row in range(_TILE * ngrp, R + C):
                    for c in range(nchunk):
                        sl = pl.ds(c * nlanes, nlanes)
                        o_v[row, sl] = x_v[row - R, sl]

            h_in = start_in(0)
            h_out = None
            for i in range(bpw):
                h_in.wait()
                if h_out is not None:
                    h_out.wait()
                shift()
                h_out = start_out(i)
                if i + 1 < bpw:
                    h_in = start_in(i + 1)
            h_out.wait()

    return sc_concat


def kernel(x, y, emb_table):
    B, C = x.shape[0], x.shape[1]
    R, D = emb_table.shape
    x3 = x.reshape(B, C, D)
    idx = y.reshape(-1).astype(jnp.int32)
    pad = jnp.broadcast_to(idx[:1], (_TILE - R,))
    idx8 = jnp.concatenate([idx, pad])
    return _build(B, C, D, R)(x3, idx8, emb_table)
